# baseline (speedup 1.0000x reference)
import functools

import jax
import jax.numpy as jnp
from jax import lax
from jax.experimental import pallas as pl
from jax.experimental.pallas import tpu as pltpu
from jax.experimental.pallas import tpu_sc as plsc

N_NODES = 10000
FDIM = 128
NGROUPS = 64

NCORES = 2
NSUB = 16
NTILES = NCORES * NSUB
CHUNK = 128
ACC_ROWS = 10240
ZCHUNKS_PER_TILE = (ACC_ROWS // NSUB) // CHUNK
OUT_MAIN = 624
OUT_TAIL = N_NODES - NSUB * OUT_MAIN


def _seg_sum_sc(x, src2d, dst2d, zeros_blk, chunks_per_tile):
    mesh = plsc.VectorSubcoreMesh(core_axis_name="c", subcore_axis_name="s")

    @functools.partial(
        pl.kernel,
        out_type=jax.ShapeDtypeStruct((NCORES, N_NODES, FDIM), jnp.float32),
        mesh=mesh,
        scratch_types=[
            pltpu.VMEM((chunks_per_tile, CHUNK), jnp.int32),
            pltpu.VMEM((chunks_per_tile, CHUNK), jnp.int32),
            pltpu.VMEM((CHUNK, FDIM), jnp.float32),
            pltpu.VMEM_SHARED((ACC_ROWS, FDIM), jnp.float32),
        ],
    )
    def k(x_hbm, src_hbm, dst_hbm, z_hbm, out_hbm, idx_s, idx_d, rows, acc):
        c = lax.axis_index("c")
        s = lax.axis_index("s")
        w = c * NSUB + s

        @pl.loop(0, ZCHUNKS_PER_TILE)
        def _(kk):
            pltpu.sync_copy(
                z_hbm, acc.at[pl.ds(s * (ACC_ROWS // NSUB) + kk * CHUNK, CHUNK)])

        pltpu.sync_copy(src_hbm.at[pl.ds(w * chunks_per_tile, chunks_per_tile)],
                        idx_s)
        pltpu.sync_copy(dst_hbm.at[pl.ds(w * chunks_per_tile, chunks_per_tile)],
                        idx_d)
        plsc.subcore_barrier()

        @pl.loop(0, chunks_per_tile)
        def _(j):
            pltpu.sync_copy(x_hbm.at[idx_s.at[j]], rows)
            pltpu.sync_copy(rows, acc.at[idx_d.at[j]], add=True)

        plsc.subcore_barrier()

        pltpu.sync_copy(acc.at[pl.ds(s * OUT_MAIN, OUT_MAIN)],
                        out_hbm.at[c, pl.ds(s * OUT_MAIN, OUT_MAIN)])

        @pl.when(s == NSUB - 1)
        def _():
            pltpu.sync_copy(acc.at[pl.ds(NSUB * OUT_MAIN, OUT_TAIL)],
                            out_hbm.at[c, pl.ds(NSUB * OUT_MAIN, OUT_TAIL)])

    return k(x, src2d, dst2d, zeros_blk)


_INV_N = float(__import__("numpy").float32(1.0) / __import__("numpy").float32(N_NODES))


def _xla_stats(hr, ncols):
    half = N_NODES // 2
    tiles = half // 8

    def win(base, f):
        def body(t, acc):
            tile = hr[pl.ds(base + t * 8, 8), :ncols]
            return acc + f(tile)
        acc = lax.fori_loop(0, tiles, body, jnp.zeros((8, ncols), jnp.float32))
        a = acc[:4] + acc[4:]
        a = a[:2] + a[2:]
        return a[0:1] + a[1:2]

    ident = lambda t: t
    m = (win(0, ident) + win(half, ident)) * _INV_N
    sq = lambda t: (t - m) * (t - m)
    v = (win(0, sq) + win(half, sq)) * _INV_N
    return m, v


def _mlp_body(x_ref, p0_ref, p1_ref, eps_ref, w1_ref, b1_ref, g1_ref, bb1_ref,
              w2_ref, b2_ref, g2_ref, bb2_ref, o_ref, h_ref):
    u = (1.0 + eps_ref[0, 0]) * x_ref[...] + p0_ref[...] + p1_ref[...]
    h_ref[...] = jnp.dot(u, w1_ref[...], preferred_element_type=jnp.float32) + b1_ref[...]
    m, v = _xla_stats(h_ref, 2 * FDIM)
    h = g1_ref[...] * (h_ref[...] - m) * lax.rsqrt(v + 1e-5) + bb1_ref[...]
    h = jnp.maximum(h, 0.0)
    h = jnp.dot(h, w2_ref[...], preferred_element_type=jnp.float32) + b2_ref[...]
    h_ref[:, :FDIM] = h
    m2, v2 = _xla_stats(h_ref, FDIM)
    h = g2_ref[...] * (h - m2) * lax.rsqrt(v2 + 1e-5) + bb2_ref[...]
    o_ref[...] = jnp.maximum(h, 0.0)


def _mlp_tc(x, p0, p1, lp):
    eps = lp["eps"].reshape(1, 1)
    args = (x, p0, p1, eps,
            lp["lin1"]["W"], lp["lin1"]["b"].reshape(1, -1),
            lp["g1"].reshape(1, -1), lp["b1"].reshape(1, -1),
            lp["lin2"]["W"], lp["lin2"]["b"].reshape(1, -1),
            lp["g_out"].reshape(1, -1), lp["b_out"].reshape(1, -1))
    return pl.pallas_call(
        _mlp_body,
        out_shape=jax.ShapeDtypeStruct((N_NODES, FDIM), jnp.float32),
        scratch_shapes=[pltpu.VMEM((N_NODES, 2 * FDIM), jnp.float32)],
    )(*args)


def _pool_cls_body(x_ref, brow_ref, bcol_ref, w1_ref, c1_ref, g_ref, bb_ref,
                   w2_ref, c2_ref, w3_ref, c3_ref, o_ref, mx_ref):
    xx = x_ref[...]
    gid = lax.broadcasted_iota(jnp.int32, (NGROUPS, N_NODES), 0)
    mt = (brow_ref[...] == gid).astype(jnp.float32)
    s = jnp.dot(mt, xx, preferred_element_type=jnp.float32,
                precision=lax.Precision.HIGHEST)
    cnt = jnp.sum(mt, axis=1, keepdims=True)
    mean = s / jnp.maximum(cnt, 1.0)
    bcol = bcol_ref[...]

    def mx_step(g, carry):
        mg = jnp.max(jnp.where(bcol == g, xx, -jnp.inf), axis=0)
        mx_ref[pl.ds(g, 1), :] = mg[None]
        return carry

    lax.fori_loop(0, NGROUPS, mx_step, 0)
    mx = jnp.where(cnt > 0.0, mx_ref[...], 0.0)
    z = jnp.concatenate([s, mean, mx], axis=1)
    z = jnp.dot(z, w1_ref[...], preferred_element_type=jnp.float32) + c1_ref[...]
    m = jnp.mean(z, axis=0, keepdims=True)
    v = jnp.mean((z - m) * (z - m), axis=0, keepdims=True)
    z = g_ref[...] * (z - m) * lax.rsqrt(v + 1e-5) + bb_ref[...]
    z = jnp.maximum(z, 0.0)
    z = jnp.dot(z, w2_ref[...], preferred_element_type=jnp.float32) + c2_ref[...]
    z = jnp.maximum(z, 0.0)
    o_ref[...] = (jnp.dot(z, w3_ref[...], preferred_element_type=jnp.float32)
                  + c3_ref[...])


def _pool_cls_tc(x, brow, bcol, cls):
    args = (x, brow, bcol,
            cls["l1"]["W"], cls["l1"]["b"].reshape(1, -1),
            cls["g"].reshape(1, -1), cls["b"].reshape(1, -1),
            cls["l2"]["W"], cls["l2"]["b"].reshape(1, -1),
            cls["l3"]["W"], cls["l3"]["b"].reshape(1, -1))
    nc = cls["l3"]["W"].shape[1]
    return pl.pallas_call(
        _pool_cls_body,
        out_shape=jax.ShapeDtypeStruct((NGROUPS, nc), jnp.float32),
        scratch_shapes=[pltpu.VMEM((NGROUPS, FDIM), jnp.float32)],
    )(*args)


def kernel(x, edge_index, batch, params):
    src = edge_index[0].astype(jnp.int32)
    dst = edge_index[1].astype(jnp.int32)
    e = src.shape[0]
    chunks = -(-e // CHUNK)
    cpt = -(-chunks // NTILES)
    cpt = -(-cpt // 8) * 8
    epad = cpt * NTILES * CHUNK
    src_p = jnp.concatenate([src, jnp.zeros((epad - e,), jnp.int32)])
    dst_p = jnp.concatenate([dst, jnp.full((epad - e,), N_NODES, jnp.int32)])
    src2d = src_p.reshape(-1, CHUNK)
    dst2d = dst_p.reshape(-1, CHUNK)
    zeros_blk = jnp.zeros((CHUNK, FDIM), jnp.float32)

    xcur = x
    for lp in params["layers"]:
        parts = _seg_sum_sc(xcur, src2d, dst2d, zeros_blk, cpt)
        xcur = _mlp_tc(xcur, parts[0], parts[1], lp)

    brow = batch.astype(jnp.int32).reshape(1, N_NODES)
    bcol = batch.astype(jnp.int32).reshape(N_NODES, 1)
    return _pool_cls_tc(xcur, brow, bcol, params["cls"])


# speedup vs baseline: 1.0722x; 1.0722x over previous
import functools

import jax
import jax.numpy as jnp
from jax import lax
from jax.experimental import pallas as pl
from jax.experimental.pallas import tpu as pltpu
from jax.experimental.pallas import tpu_sc as plsc

N_NODES = 10000
FDIM = 128
NGROUPS = 64

NCORES = 2
NSUB = 16
NTILES = NCORES * NSUB
CHUNK = 128
ACC_ROWS = 10240
ZCHUNKS_PER_TILE = (ACC_ROWS // NSUB) // CHUNK
OUT_MAIN = 624
OUT_TAIL = N_NODES - NSUB * OUT_MAIN


def _seg_sum_sc(x, src2d, dst2d, zeros_blk, chunks_per_tile):
    mesh = plsc.VectorSubcoreMesh(core_axis_name="c", subcore_axis_name="s")

    @functools.partial(
        pl.kernel,
        out_type=jax.ShapeDtypeStruct((NCORES, N_NODES, FDIM), jnp.float32),
        mesh=mesh,
        scratch_types=[
            pltpu.VMEM((chunks_per_tile // 2, CHUNK), jnp.int32),
            pltpu.VMEM((chunks_per_tile // 2, CHUNK), jnp.int32),
            pltpu.VMEM((CHUNK, FDIM), jnp.float32),
            pltpu.VMEM((CHUNK, FDIM), jnp.float32),
            pltpu.VMEM_SHARED((ACC_ROWS, FDIM), jnp.float32),
            pltpu.SemaphoreType.DMA,
            pltpu.SemaphoreType.DMA,
        ],
    )
    def k(x_hbm, src_hbm, dst_hbm, z_hbm, out_hbm, idx_s, idx_d,
          rows0, rows1, acc, sem_a, sem_b):
        c = lax.axis_index("c")
        s = lax.axis_index("s")
        w = c * NSUB + s

        @pl.loop(0, ZCHUNKS_PER_TILE)
        def _(kk):
            pltpu.sync_copy(
                z_hbm, acc.at[pl.ds(s * (ACC_ROWS // NSUB) + kk * CHUNK, CHUNK)])

        plsc.subcore_barrier()

        cph = chunks_per_tile // 2

        @pl.loop(0, 2)
        def _(half):
            base = w * chunks_per_tile + half * cph
            pltpu.sync_copy(src_hbm.at[pl.ds(base, cph)], idx_s)
            pltpu.sync_copy(dst_hbm.at[pl.ds(base, cph)], idx_d)
            pltpu.async_copy(x_hbm.at[idx_s.at[0]], rows0, sem_a)

            @pl.loop(0, cph // 2)
            def _(jj):
                j0 = 2 * jj
                pltpu.make_async_copy(x_hbm.at[idx_s.at[j0]], rows0, sem_a).wait()
                pltpu.async_copy(x_hbm.at[idx_s.at[j0 + 1]], rows1, sem_b)
                pltpu.sync_copy(rows0, acc.at[idx_d.at[j0]], add=True)
                pltpu.make_async_copy(x_hbm.at[idx_s.at[j0 + 1]], rows1,
                                      sem_b).wait()

                @pl.when(j0 + 2 < cph)
                def _():
                    pltpu.async_copy(x_hbm.at[idx_s.at[j0 + 2]], rows0, sem_a)

                pltpu.sync_copy(rows1, acc.at[idx_d.at[j0 + 1]], add=True)

        plsc.subcore_barrier()

        pltpu.sync_copy(acc.at[pl.ds(s * OUT_MAIN, OUT_MAIN)],
                        out_hbm.at[c, pl.ds(s * OUT_MAIN, OUT_MAIN)])

        @pl.when(s == NSUB - 1)
        def _():
            pltpu.sync_copy(acc.at[pl.ds(NSUB * OUT_MAIN, OUT_TAIL)],
                            out_hbm.at[c, pl.ds(NSUB * OUT_MAIN, OUT_TAIL)])

    return k(x, src2d, dst2d, zeros_blk)


_INV_N = float(__import__("numpy").float32(1.0) / __import__("numpy").float32(N_NODES))


def _xla_stats(hr, ncols):
    half = N_NODES // 2
    tiles = half // 8

    def win(base, f):
        def body(t, acc):
            tile = hr[pl.ds(base + t * 8, 8), :ncols]
            return acc + f(tile)
        acc = lax.fori_loop(0, tiles, body, jnp.zeros((8, ncols), jnp.float32))
        a = acc[:4] + acc[4:]
        a = a[:2] + a[2:]
        return a[0:1] + a[1:2]

    ident = lambda t: t
    m = (win(0, ident) + win(half, ident)) * _INV_N
    sq = lambda t: (t - m) * (t - m)
    v = (win(0, sq) + win(half, sq)) * _INV_N
    return m, v


def _mlp_body(x_ref, p0_ref, p1_ref, eps_ref, w1_ref, b1_ref, g1_ref, bb1_ref,
              w2_ref, b2_ref, g2_ref, bb2_ref, o_ref, h_ref):
    u = (1.0 + eps_ref[0, 0]) * x_ref[...] + p0_ref[...] + p1_ref[...]
    h_ref[...] = jnp.dot(u, w1_ref[...], preferred_element_type=jnp.float32) + b1_ref[...]
    m, v = _xla_stats(h_ref, 2 * FDIM)
    h = g1_ref[...] * (h_ref[...] - m) * lax.rsqrt(v + 1e-5) + bb1_ref[...]
    h = jnp.maximum(h, 0.0)
    h = jnp.dot(h, w2_ref[...], preferred_element_type=jnp.float32) + b2_ref[...]
    h_ref[:, :FDIM] = h
    m2, v2 = _xla_stats(h_ref, FDIM)
    h = g2_ref[...] * (h - m2) * lax.rsqrt(v2 + 1e-5) + bb2_ref[...]
    o_ref[...] = jnp.maximum(h, 0.0)


def _mlp_tc(x, p0, p1, lp):
    eps = lp["eps"].reshape(1, 1)
    args = (x, p0, p1, eps,
            lp["lin1"]["W"], lp["lin1"]["b"].reshape(1, -1),
            lp["g1"].reshape(1, -1), lp["b1"].reshape(1, -1),
            lp["lin2"]["W"], lp["lin2"]["b"].reshape(1, -1),
            lp["g_out"].reshape(1, -1), lp["b_out"].reshape(1, -1))
    return pl.pallas_call(
        _mlp_body,
        out_shape=jax.ShapeDtypeStruct((N_NODES, FDIM), jnp.float32),
        scratch_shapes=[pltpu.VMEM((N_NODES, 2 * FDIM), jnp.float32)],
    )(*args)


def _pool_cls_body(x_ref, brow_ref, bcol_ref, w1_ref, c1_ref, g_ref, bb_ref,
                   w2_ref, c2_ref, w3_ref, c3_ref, o_ref, mx_ref):
    xx = x_ref[...]
    gid = lax.broadcasted_iota(jnp.int32, (NGROUPS, N_NODES), 0)
    mt = (brow_ref[...] == gid).astype(jnp.float32)
    s = jnp.dot(mt, xx, preferred_element_type=jnp.float32,
                precision=lax.Precision.HIGHEST)
    cnt = jnp.sum(mt, axis=1, keepdims=True)
    mean = s / jnp.maximum(cnt, 1.0)
    bcol = bcol_ref[...]

    def mx_step(g, carry):
        mg = jnp.max(jnp.where(bcol == g, xx, -jnp.inf), axis=0)
        mx_ref[pl.ds(g, 1), :] = mg[None]
        return carry

    lax.fori_loop(0, NGROUPS, mx_step, 0)
    mx = jnp.where(cnt > 0.0, mx_ref[...], 0.0)
    z = jnp.concatenate([s, mean, mx], axis=1)
    z = jnp.dot(z, w1_ref[...], preferred_element_type=jnp.float32) + c1_ref[...]
    m = jnp.mean(z, axis=0, keepdims=True)
    v = jnp.mean((z - m) * (z - m), axis=0, keepdims=True)
    z = g_ref[...] * (z - m) * lax.rsqrt(v + 1e-5) + bb_ref[...]
    z = jnp.maximum(z, 0.0)
    z = jnp.dot(z, w2_ref[...], preferred_element_type=jnp.float32) + c2_ref[...]
    z = jnp.maximum(z, 0.0)
    o_ref[...] = (jnp.dot(z, w3_ref[...], preferred_element_type=jnp.float32)
                  + c3_ref[...])


def _pool_cls_tc(x, brow, bcol, cls):
    args = (x, brow, bcol,
            cls["l1"]["W"], cls["l1"]["b"].reshape(1, -1),
            cls["g"].reshape(1, -1), cls["b"].reshape(1, -1),
            cls["l2"]["W"], cls["l2"]["b"].reshape(1, -1),
            cls["l3"]["W"], cls["l3"]["b"].reshape(1, -1))
    nc = cls["l3"]["W"].shape[1]
    return pl.pallas_call(
        _pool_cls_body,
        out_shape=jax.ShapeDtypeStruct((NGROUPS, nc), jnp.float32),
        scratch_shapes=[pltpu.VMEM((NGROUPS, FDIM), jnp.float32)],
    )(*args)


def kernel(x, edge_index, batch, params):
    src = edge_index[0].astype(jnp.int32)
    dst = edge_index[1].astype(jnp.int32)
    e = src.shape[0]
    chunks = -(-e // CHUNK)
    cpt = -(-chunks // NTILES)
    cpt = -(-cpt // 8) * 8
    epad = cpt * NTILES * CHUNK
    src_p = jnp.concatenate([src, jnp.zeros((epad - e,), jnp.int32)])
    dst_p = jnp.concatenate([dst, jnp.full((epad - e,), N_NODES, jnp.int32)])
    src2d = src_p.reshape(-1, CHUNK)
    dst2d = dst_p.reshape(-1, CHUNK)
    zeros_blk = jnp.zeros((CHUNK, FDIM), jnp.float32)

    xcur = x
    for lp in params["layers"]:
        parts = _seg_sum_sc(xcur, src2d, dst2d, zeros_blk, cpt)
        xcur = _mlp_tc(xcur, parts[0], parts[1], lp)

    brow = batch.astype(jnp.int32).reshape(1, N_NODES)
    bcol = batch.astype(jnp.int32).reshape(N_NODES, 1)
    return _pool_cls_tc(xcur, brow, bcol, params["cls"])


# speedup vs baseline: 2.7348x; 2.5505x over previous
import functools

import jax
import jax.numpy as jnp
from jax import lax
from jax.experimental import pallas as pl
from jax.experimental.pallas import tpu as pltpu
from jax.experimental.pallas import tpu_sc as plsc

N_NODES = 10000
FDIM = 128
NGROUPS = 64

NCORES = 2
NSUB = 16
NTILES = NCORES * NSUB
CHUNK = 128
ACC_ROWS = 10240
ZCHUNKS_PER_TILE = (ACC_ROWS // NSUB) // CHUNK
OUT_MAIN = 624
OUT_TAIL = N_NODES - NSUB * OUT_MAIN


def _seg_sum_sc(x, src2d, dst2d, zeros_blk, chunks_per_tile):
    mesh = plsc.VectorSubcoreMesh(core_axis_name="c", subcore_axis_name="s")

    @functools.partial(
        pl.kernel,
        out_type=jax.ShapeDtypeStruct((NCORES, N_NODES, FDIM), jnp.float32),
        mesh=mesh,
        scratch_types=[
            pltpu.VMEM((chunks_per_tile // 2, CHUNK), jnp.int32),
            pltpu.VMEM((chunks_per_tile // 2, CHUNK), jnp.int32),
            pltpu.VMEM((CHUNK, FDIM), jnp.float32),
            pltpu.VMEM((CHUNK, FDIM), jnp.float32),
            pltpu.VMEM_SHARED((ACC_ROWS, FDIM), jnp.float32),
            pltpu.SemaphoreType.DMA,
            pltpu.SemaphoreType.DMA,
        ],
    )
    def k(x_hbm, src_hbm, dst_hbm, z_hbm, out_hbm, idx_s, idx_d,
          rows0, rows1, acc, sem_a, sem_b):
        c = lax.axis_index("c")
        s = lax.axis_index("s")
        w = c * NSUB + s

        @pl.loop(0, ZCHUNKS_PER_TILE)
        def _(kk):
            pltpu.sync_copy(
                z_hbm, acc.at[pl.ds(s * (ACC_ROWS // NSUB) + kk * CHUNK, CHUNK)])

        plsc.subcore_barrier()

        cph = chunks_per_tile // 2

        @pl.loop(0, 2)
        def _(half):
            base = w * chunks_per_tile + half * cph
            pltpu.sync_copy(src_hbm.at[pl.ds(base, cph)], idx_s)
            pltpu.sync_copy(dst_hbm.at[pl.ds(base, cph)], idx_d)
            pltpu.async_copy(x_hbm.at[idx_s.at[0]], rows0, sem_a)

            @pl.loop(0, cph // 2)
            def _(jj):
                j0 = 2 * jj
                pltpu.make_async_copy(x_hbm.at[idx_s.at[j0]], rows0, sem_a).wait()
                pltpu.async_copy(x_hbm.at[idx_s.at[j0 + 1]], rows1, sem_b)
                pltpu.sync_copy(rows0, acc.at[idx_d.at[j0]], add=True)
                pltpu.make_async_copy(x_hbm.at[idx_s.at[j0 + 1]], rows1,
                                      sem_b).wait()

                @pl.when(j0 + 2 < cph)
                def _():
                    pltpu.async_copy(x_hbm.at[idx_s.at[j0 + 2]], rows0, sem_a)

                pltpu.sync_copy(rows1, acc.at[idx_d.at[j0 + 1]], add=True)

        plsc.subcore_barrier()

        pltpu.sync_copy(acc.at[pl.ds(s * OUT_MAIN, OUT_MAIN)],
                        out_hbm.at[c, pl.ds(s * OUT_MAIN, OUT_MAIN)])

        @pl.when(s == NSUB - 1)
        def _():
            pltpu.sync_copy(acc.at[pl.ds(NSUB * OUT_MAIN, OUT_TAIL)],
                            out_hbm.at[c, pl.ds(NSUB * OUT_MAIN, OUT_TAIL)])

    return k(x, src2d, dst2d, zeros_blk)


_INV_N = float(__import__("numpy").float32(1.0) / __import__("numpy").float32(N_NODES))


def _xla_stats(hr, ncols):
    half = N_NODES // 2
    tiles = half // 8

    def win(base, f):
        def body(t, acc):
            tile = hr[pl.ds(base + t * 8, 8), :ncols]
            return acc + f(tile)
        acc = lax.fori_loop(0, tiles, body, jnp.zeros((8, ncols), jnp.float32))
        a = acc[:4] + acc[4:]
        a = a[:2] + a[2:]
        return a[0:1] + a[1:2]

    ident = lambda t: t
    m = (win(0, ident) + win(half, ident)) * _INV_N
    sq = lambda t: (t - m) * (t - m)
    v = (win(0, sq) + win(half, sq)) * _INV_N
    return m, v


def _mlp_body(x_ref, p0_ref, p1_ref, eps_ref, w1_ref, b1_ref, g1_ref, bb1_ref,
              w2_ref, b2_ref, g2_ref, bb2_ref, o_ref, h_ref):
    u = (1.0 + eps_ref[0, 0]) * x_ref[...] + p0_ref[...] + p1_ref[...]
    h_ref[...] = jnp.dot(u, w1_ref[...], preferred_element_type=jnp.float32) + b1_ref[...]
    m, v = _xla_stats(h_ref, 2 * FDIM)
    h = g1_ref[...] * (h_ref[...] - m) * lax.rsqrt(v + 1e-5) + bb1_ref[...]
    h = jnp.maximum(h, 0.0)
    h = jnp.dot(h, w2_ref[...], preferred_element_type=jnp.float32) + b2_ref[...]
    h_ref[:, :FDIM] = h
    m2, v2 = _xla_stats(h_ref, FDIM)
    h = g2_ref[...] * (h - m2) * lax.rsqrt(v2 + 1e-5) + bb2_ref[...]
    o_ref[...] = jnp.maximum(h, 0.0)


def _mlp_tc(x, p0, p1, lp):
    eps = lp["eps"].reshape(1, 1)
    args = (x, p0, p1, eps,
            lp["lin1"]["W"], lp["lin1"]["b"].reshape(1, -1),
            lp["g1"].reshape(1, -1), lp["b1"].reshape(1, -1),
            lp["lin2"]["W"], lp["lin2"]["b"].reshape(1, -1),
            lp["g_out"].reshape(1, -1), lp["b_out"].reshape(1, -1))
    return pl.pallas_call(
        _mlp_body,
        out_shape=jax.ShapeDtypeStruct((N_NODES, FDIM), jnp.float32),
        scratch_shapes=[pltpu.VMEM((N_NODES, 2 * FDIM), jnp.float32)],
    )(*args)


def _pool_cls_body(x_ref, brow_ref, bcol_ref, w1_ref, c1_ref, g_ref, bb_ref,
                   w2_ref, c2_ref, w3_ref, c3_ref, o_ref, mx_ref):
    xx = x_ref[...]
    gid = lax.broadcasted_iota(jnp.int32, (NGROUPS, N_NODES), 0)
    mt = (brow_ref[...] == gid).astype(jnp.float32)
    s = jnp.dot(mt, xx, preferred_element_type=jnp.float32,
                precision=lax.Precision.HIGHEST)
    cnt = jnp.sum(mt, axis=1, keepdims=True)
    mean = s / jnp.maximum(cnt, 1.0)
    bcol = bcol_ref[...]

    def mx_step(g, carry):
        mg = jnp.max(jnp.where(bcol == g, xx, -jnp.inf), axis=0)
        mx_ref[pl.ds(g, 1), :] = mg[None]
        return carry

    lax.fori_loop(0, NGROUPS, mx_step, 0)
    mx = jnp.where(cnt > 0.0, mx_ref[...], 0.0)
    z = jnp.concatenate([s, mean, mx], axis=1)
    z = jnp.dot(z, w1_ref[...], preferred_element_type=jnp.float32) + c1_ref[...]
    m = jnp.mean(z, axis=0, keepdims=True)
    v = jnp.mean((z - m) * (z - m), axis=0, keepdims=True)
    z = g_ref[...] * (z - m) * lax.rsqrt(v + 1e-5) + bb_ref[...]
    z = jnp.maximum(z, 0.0)
    z = jnp.dot(z, w2_ref[...], preferred_element_type=jnp.float32) + c2_ref[...]
    z = jnp.maximum(z, 0.0)
    o_ref[...] = (jnp.dot(z, w3_ref[...], preferred_element_type=jnp.float32)
                  + c3_ref[...])


def _pool_cls_tc(x, brow, bcol, cls):
    args = (x, brow, bcol,
            cls["l1"]["W"], cls["l1"]["b"].reshape(1, -1),
            cls["g"].reshape(1, -1), cls["b"].reshape(1, -1),
            cls["l2"]["W"], cls["l2"]["b"].reshape(1, -1),
            cls["l3"]["W"], cls["l3"]["b"].reshape(1, -1))
    nc = cls["l3"]["W"].shape[1]
    return pl.pallas_call(
        _pool_cls_body,
        out_shape=jax.ShapeDtypeStruct((NGROUPS, nc), jnp.float32),
        scratch_shapes=[pltpu.VMEM((NGROUPS, FDIM), jnp.float32)],
    )(*args)


def kernel(x, edge_index, batch, params):
    src = edge_index[0].astype(jnp.int32)
    dst = edge_index[1].astype(jnp.int32)
    e = src.shape[0]
    chunks = -(-e // CHUNK)
    cpt = -(-chunks // NTILES)
    cpt = -(-cpt // 8) * 8
    epad = cpt * NTILES * CHUNK
    pad_ids = jnp.arange(epad - e, dtype=jnp.int32)
    src_p = jnp.concatenate([src, pad_ids % N_NODES])
    dst_p = jnp.concatenate(
        [dst, N_NODES + pad_ids % (ACC_ROWS - N_NODES)])
    src2d = src_p.reshape(-1, CHUNK)
    dst2d = dst_p.reshape(-1, CHUNK)
    zeros_blk = jnp.zeros((CHUNK, FDIM), jnp.float32)

    xcur = x
    for lp in params["layers"]:
        parts = _seg_sum_sc(xcur, src2d, dst2d, zeros_blk, cpt)
        xcur = _mlp_tc(xcur, parts[0], parts[1], lp)

    brow = batch.astype(jnp.int32).reshape(1, N_NODES)
    bcol = batch.astype(jnp.int32).reshape(N_NODES, 1)
    return _pool_cls_tc(xcur, brow, bcol, params["cls"])
